# baseline (speedup 1.0000x reference)
import jax
import jax.numpy as jnp
from jax import lax
from jax.experimental import pallas as pl
from jax.experimental.pallas import tpu as pltpu
from jax.experimental.pallas import tpu_sc as plsc

NC, NS = 2, 16
NW = NC * NS
_N, _E, _D = 10000, 320000, 128
_M = _E + _N
_S = 20640
KG = 128
NCHM = 162
NA = _N + 16
SAC = NA
ACC_ROWS = NA + 8
RB = 8
NZB = ACC_ROWS // RB
NRB = NA // RB

_mesh = plsc.VectorSubcoreMesh(core_axis_name="c", subcore_axis_name="s")


def _agg_body(h_hbm, tbl_hbm, combo_hbm, z_hbm, out_hbm,
              acc, idx3, rowsA, rowsB, rb_v, gsem):
    cid = lax.axis_index("c")
    sid = lax.axis_index("s")

    pltpu.sync_copy(z_hbm, rb_v)

    @pl.loop(sid, NZB, step=NS)
    def _(j):
        pltpu.sync_copy(rb_v, acc.at[pl.ds(j * RB, RB)])

    plsc.subcore_barrier()

    @pl.when(cid == 0)
    def _():
        @pl.loop(0, NCHM)
        def _(j):
            pltpu.sync_copy(combo_hbm.at[sid, j], idx3)
            pltpu.async_copy(h_hbm.at[idx3.at[0]], rowsA, gsem).wait()
            pltpu.async_copy(tbl_hbm.at[idx3.at[1]], rowsB, gsem).wait()

            @pl.loop(0, KG)
            def _(e):
                for u in range(_D // 16):
                    sl = pl.ds(u * 16, 16)
                    rowsA[e, sl] = rowsA[e, sl] + rowsB[e, sl]

            pltpu.sync_copy(rowsA, acc.at[idx3.at[2]], add=True)

    plsc.subcore_barrier()

    @pl.loop(sid, NRB, step=NS)
    def _(j):
        pltpu.sync_copy(acc.at[pl.ds(j * RB, RB)], rb_v)
        pltpu.sync_copy(rb_v, out_hbm.at[cid, pl.ds(j * RB, RB)])


_agg_call = pl.kernel(
    _agg_body,
    out_type=jax.ShapeDtypeStruct((NC, NA, _D), jnp.float32),
    mesh=_mesh,
    scratch_types=[
        pltpu.VMEM_SHARED((ACC_ROWS, _D), jnp.float32),
        pltpu.VMEM((3, KG), jnp.int32),
        pltpu.VMEM((KG, _D), jnp.float32),
        pltpu.VMEM((KG, _D), jnp.float32),
        pltpu.VMEM((RB, _D), jnp.float32),
        pltpu.SemaphoreType.DMA,
    ],
)


def kernel(x, edge_index, edge_attr, xe1, xe2, xe3, xe4, xe5, xe6,
           W1, b1, W2, b2, ee1, ee2, gamma, beta):
    L = W1.shape[0]
    i32 = jnp.int32

    loops = jnp.arange(_N, dtype=edge_index.dtype)
    src_f = jnp.concatenate([edge_index[0], loops])
    dst_f = jnp.concatenate([edge_index[1], loops])
    a0 = jnp.concatenate([edge_attr[:, 0], jnp.full((_N,), 4, edge_attr.dtype)])
    a1 = jnp.concatenate([edge_attr[:, 1], jnp.zeros((_N,), edge_attr.dtype)])
    trow_f = jnp.where(a0 == 4, 9, a0 * 3 + a1).astype(i32)

    perm = jnp.argsort(dst_f, stable=True)
    ssrc = src_f[perm].astype(i32)
    strow = trow_f[perm]
    sdst = dst_f[perm].astype(i32)
    rowptr = jnp.searchsorted(sdst, jnp.arange(_N + 1, dtype=i32)).astype(i32)

    pos = jnp.arange(_M, dtype=i32)
    slab = pos // _S
    nodestart = rowptr[sdst]
    vdst = jnp.where((slab > 0) & (nodestart < slab * _S),
                     _N + slab - 1, sdst).astype(i32)

    tile_of = pos // _S
    local = pos - tile_of * _S
    flat = tile_of * (NCHM * KG) + local
    c_src = jnp.zeros((16 * NCHM * KG,), i32).at[flat].set(ssrc)
    c_trow = jnp.full((16 * NCHM * KG,), 15, i32).at[flat].set(strow)
    c_dst = jnp.full((16 * NCHM * KG,), SAC, i32).at[flat].set(vdst)
    combo = jnp.stack([c_src.reshape(16, NCHM, KG),
                       c_trow.reshape(16, NCHM, KG),
                       c_dst.reshape(16, NCHM, KG)], axis=2)

    kb = jnp.arange(15, dtype=i32)
    bpos = (kb + 1) * _S
    bnode = sdst[jnp.clip(bpos, 0, _M - 1)]
    issplit = (rowptr[bnode] < bpos) & (rowptr[bnode + 1] > bpos)

    z_d = jnp.zeros((RB, _D), jnp.float32)

    i1 = jnp.array([0, 0, 0, 1, 1, 1, 2, 2, 2], i32)
    i2 = jnp.array([0, 1, 2, 0, 1, 2, 0, 1, 2], i32)
    tbl = jnp.concatenate(
        [ee1[:, i1, :] + ee2[:, i2, :],
         (ee1[:, 4, :] + ee2[:, 0, :])[:, None, :],
         jnp.zeros((L, 6, _D), jnp.float32)], axis=1)

    h = (xe1[x[:, 0]] + xe2[x[:, 1]] + xe3[x[:, 2]]
         + xe4[x[:, 3]] + xe5[x[:, 4]] + xe6[x[:, 5]])

    for l in range(L):
        p = _agg_call(h, tbl[l], combo, z_d)
        afull = p[0] + p[1]
        aggr = afull[:_N]
        clones = jnp.where(issplit[:, None], afull[_N:_N + 15], 0.0)
        aggr = aggr.at[bnode].add(clones)
        hid = jax.nn.relu(aggr @ W1[l] + b1[l])
        hnew = hid @ W2[l] + b2[l]
        mean = jnp.mean(hnew, axis=0)
        var = jnp.var(hnew, axis=0)
        hnew = (hnew - mean) / jnp.sqrt(var + 1e-5) * gamma[l] + beta[l]
        if l < L - 1:
            hnew = jax.nn.relu(hnew)
        h = hnew
    return h


# speedup vs baseline: 1.0010x; 1.0010x over previous
import jax
import jax.numpy as jnp
from jax import lax
from jax.experimental import pallas as pl
from jax.experimental.pallas import tpu as pltpu
from jax.experimental.pallas import tpu_sc as plsc

NC, NS = 2, 16
NW = NC * NS
_N, _E, _D = 10000, 320000, 128
_M = _E + _N
_S = 20640
KG = 128
NCHM = 162
NA = _N + 16
SAC = NA
ACC_ROWS = NA + 8
RB = 8
NZB = ACC_ROWS // RB
NRB = NA // RB

_mesh = plsc.VectorSubcoreMesh(core_axis_name="c", subcore_axis_name="s")


def _agg_body(h_hbm, tbl_hbm, combo_hbm, z_hbm, out_hbm,
              acc, idx3a, idx3b, rowsA, rowsB, rb_v, gsem, gsem2):
    cid = lax.axis_index("c")
    sid = lax.axis_index("s")

    pltpu.sync_copy(z_hbm, rb_v)

    @pl.loop(sid, NZB, step=NS)
    def _(j):
        pltpu.sync_copy(rb_v, acc.at[pl.ds(j * RB, RB)])

    plsc.subcore_barrier()

    @pl.when(cid == 0)
    def _():
        idx = (idx3a, idx3b)
        pltpu.sync_copy(combo_hbm.at[sid, 0], idx3a)

        @pl.loop(0, NCHM, step=2)
        def _(j):
            for b in range(2):
                jj = j + b
                cur = idx[b]
                nxt = idx[1 - b]
                dA = pltpu.async_copy(h_hbm.at[cur.at[0]], rowsA, gsem)
                dB = pltpu.async_copy(tbl_hbm.at[cur.at[1]], rowsB, gsem2)

                @pl.when(jj + 1 < NCHM)
                def _():
                    pltpu.sync_copy(combo_hbm.at[sid, jj + 1], nxt)

                dA.wait()
                dB.wait()

                @pl.loop(0, KG, unroll=8)
                def _(e):
                    for u in range(_D // 16):
                        sl = pl.ds(u * 16, 16)
                        rowsA[e, sl] = rowsA[e, sl] + rowsB[e, sl]

                pltpu.sync_copy(rowsA, acc.at[cur.at[2]], add=True)

    plsc.subcore_barrier()

    @pl.loop(sid, NRB, step=NS)
    def _(j):
        pltpu.sync_copy(acc.at[pl.ds(j * RB, RB)], rb_v)
        pltpu.sync_copy(rb_v, out_hbm.at[cid, pl.ds(j * RB, RB)])


_agg_call = pl.kernel(
    _agg_body,
    out_type=jax.ShapeDtypeStruct((NC, NA, _D), jnp.float32),
    mesh=_mesh,
    scratch_types=[
        pltpu.VMEM_SHARED((ACC_ROWS, _D), jnp.float32),
        pltpu.VMEM((3, KG), jnp.int32),
        pltpu.VMEM((3, KG), jnp.int32),
        pltpu.VMEM((KG, _D), jnp.float32),
        pltpu.VMEM((KG, _D), jnp.float32),
        pltpu.VMEM((RB, _D), jnp.float32),
        pltpu.SemaphoreType.DMA,
        pltpu.SemaphoreType.DMA,
    ],
)


def kernel(x, edge_index, edge_attr, xe1, xe2, xe3, xe4, xe5, xe6,
           W1, b1, W2, b2, ee1, ee2, gamma, beta):
    L = W1.shape[0]
    i32 = jnp.int32

    loops = jnp.arange(_N, dtype=edge_index.dtype)
    src_f = jnp.concatenate([edge_index[0], loops])
    dst_f = jnp.concatenate([edge_index[1], loops])
    a0 = jnp.concatenate([edge_attr[:, 0], jnp.full((_N,), 4, edge_attr.dtype)])
    a1 = jnp.concatenate([edge_attr[:, 1], jnp.zeros((_N,), edge_attr.dtype)])
    trow_f = jnp.where(a0 == 4, 9, a0 * 3 + a1).astype(i32)

    perm = jnp.argsort(dst_f, stable=True)
    ssrc = src_f[perm].astype(i32)
    strow = trow_f[perm]
    sdst = dst_f[perm].astype(i32)
    rowptr = jnp.searchsorted(sdst, jnp.arange(_N + 1, dtype=i32)).astype(i32)

    pos = jnp.arange(_M, dtype=i32)
    slab = pos // _S
    nodestart = rowptr[sdst]
    vdst = jnp.where((slab > 0) & (nodestart < slab * _S),
                     _N + slab - 1, sdst).astype(i32)

    tile_of = pos // _S
    local = pos - tile_of * _S
    flat = tile_of * (NCHM * KG) + local
    c_src = jnp.zeros((16 * NCHM * KG,), i32).at[flat].set(ssrc)
    c_trow = jnp.full((16 * NCHM * KG,), 15, i32).at[flat].set(strow)
    c_dst = jnp.full((16 * NCHM * KG,), SAC, i32).at[flat].set(vdst)
    combo = jnp.stack([c_src.reshape(16, NCHM, KG),
                       c_trow.reshape(16, NCHM, KG),
                       c_dst.reshape(16, NCHM, KG)], axis=2)

    kb = jnp.arange(15, dtype=i32)
    bpos = (kb + 1) * _S
    bnode = sdst[jnp.clip(bpos, 0, _M - 1)]
    issplit = (rowptr[bnode] < bpos) & (rowptr[bnode + 1] > bpos)

    z_d = jnp.zeros((RB, _D), jnp.float32)

    i1 = jnp.array([0, 0, 0, 1, 1, 1, 2, 2, 2], i32)
    i2 = jnp.array([0, 1, 2, 0, 1, 2, 0, 1, 2], i32)
    tbl = jnp.concatenate(
        [ee1[:, i1, :] + ee2[:, i2, :],
         (ee1[:, 4, :] + ee2[:, 0, :])[:, None, :],
         jnp.zeros((L, 6, _D), jnp.float32)], axis=1)

    h = (xe1[x[:, 0]] + xe2[x[:, 1]] + xe3[x[:, 2]]
         + xe4[x[:, 3]] + xe5[x[:, 4]] + xe6[x[:, 5]])

    for l in range(L):
        p = _agg_call(h, tbl[l], combo, z_d)
        afull = p[0] + p[1]
        aggr = afull[:_N]
        clones = jnp.where(issplit[:, None], afull[_N:_N + 15], 0.0)
        aggr = aggr.at[bnode].add(clones)
        hid = jax.nn.relu(aggr @ W1[l] + b1[l])
        hnew = hid @ W2[l] + b2[l]
        mean = jnp.mean(hnew, axis=0)
        var = jnp.var(hnew, axis=0)
        hnew = (hnew - mean) / jnp.sqrt(var + 1e-5) * gamma[l] + beta[l]
        if l < L - 1:
            hnew = jax.nn.relu(hnew)
        h = hnew
    return h


# speedup vs baseline: 1.0067x; 1.0057x over previous
import jax
import jax.numpy as jnp
from jax import lax
from jax.experimental import pallas as pl
from jax.experimental.pallas import tpu as pltpu
from jax.experimental.pallas import tpu_sc as plsc

NC, NS = 2, 16
NW = NC * NS
_N, _E, _D = 10000, 320000, 128
_M = _E + _N
_S = 20640
KG = 128
NCHM = 162
NA = _N + 16
SAC = NA
ACC_ROWS = NA + 32
RB = 32
NZB = ACC_ROWS // RB
NRB = NA // RB

_mesh = plsc.VectorSubcoreMesh(core_axis_name="c", subcore_axis_name="s")


def _agg_body(h_hbm, tbl_hbm, combo_hbm, z_hbm, out_hbm,
              acc, idx3a, idx3b, rowsA, rowsB, rb_v, gsem, gsem2):
    cid = lax.axis_index("c")
    sid = lax.axis_index("s")

    pltpu.sync_copy(z_hbm, rb_v)

    @pl.when(cid == 0)
    def _():
        @pl.loop(sid, NZB, step=NS)
        def _(j):
            pltpu.sync_copy(rb_v, acc.at[pl.ds(j * RB, RB)])

    plsc.subcore_barrier()

    @pl.when(cid == 0)
    def _():
        idx = (idx3a, idx3b)
        pltpu.sync_copy(combo_hbm.at[sid, 0], idx3a)

        @pl.loop(0, NCHM, step=2)
        def _(j):
            for b in range(2):
                jj = j + b
                cur = idx[b]
                nxt = idx[1 - b]
                dA = pltpu.async_copy(h_hbm.at[cur.at[0]], rowsA, gsem)
                dB = pltpu.async_copy(tbl_hbm.at[cur.at[1]], rowsB, gsem2)

                @pl.when(jj + 1 < NCHM)
                def _():
                    pltpu.sync_copy(combo_hbm.at[sid, jj + 1], nxt)

                dA.wait()
                dB.wait()

                @pl.loop(0, KG, unroll=8)
                def _(e):
                    for u in range(_D // 16):
                        sl = pl.ds(u * 16, 16)
                        rowsA[e, sl] = rowsA[e, sl] + rowsB[e, sl]

                pltpu.sync_copy(rowsA, acc.at[cur.at[2]], add=True)

    plsc.subcore_barrier()

    @pl.when(cid == 0)
    def _():
        @pl.loop(sid, NRB, step=NS)
        def _(j):
            pltpu.sync_copy(acc.at[pl.ds(j * RB, RB)], rb_v)
            pltpu.sync_copy(rb_v, out_hbm.at[pl.ds(j * RB, RB)])


_agg_call = pl.kernel(
    _agg_body,
    out_type=jax.ShapeDtypeStruct((NA, _D), jnp.float32),
    mesh=_mesh,
    scratch_types=[
        pltpu.VMEM_SHARED((ACC_ROWS, _D), jnp.float32),
        pltpu.VMEM((3, KG), jnp.int32),
        pltpu.VMEM((3, KG), jnp.int32),
        pltpu.VMEM((KG, _D), jnp.float32),
        pltpu.VMEM((KG, _D), jnp.float32),
        pltpu.VMEM((RB, _D), jnp.float32),
        pltpu.SemaphoreType.DMA,
        pltpu.SemaphoreType.DMA,
    ],
)


def kernel(x, edge_index, edge_attr, xe1, xe2, xe3, xe4, xe5, xe6,
           W1, b1, W2, b2, ee1, ee2, gamma, beta):
    L = W1.shape[0]
    i32 = jnp.int32

    loops = jnp.arange(_N, dtype=edge_index.dtype)
    src_f = jnp.concatenate([edge_index[0], loops])
    dst_f = jnp.concatenate([edge_index[1], loops])
    a0 = jnp.concatenate([edge_attr[:, 0], jnp.full((_N,), 4, edge_attr.dtype)])
    a1 = jnp.concatenate([edge_attr[:, 1], jnp.zeros((_N,), edge_attr.dtype)])
    trow_f = jnp.where(a0 == 4, 9, a0 * 3 + a1).astype(i32)

    perm = jnp.argsort(dst_f, stable=True)
    ssrc = src_f[perm].astype(i32)
    strow = trow_f[perm]
    sdst = dst_f[perm].astype(i32)
    rowptr = jnp.searchsorted(sdst, jnp.arange(_N + 1, dtype=i32)).astype(i32)

    pos = jnp.arange(_M, dtype=i32)
    slab = pos // _S
    nodestart = rowptr[sdst]
    vdst = jnp.where((slab > 0) & (nodestart < slab * _S),
                     _N + slab - 1, sdst).astype(i32)

    tile_of = pos // _S
    local = pos - tile_of * _S
    flat = tile_of * (NCHM * KG) + local
    c_src = jnp.zeros((16 * NCHM * KG,), i32).at[flat].set(ssrc)
    c_trow = jnp.full((16 * NCHM * KG,), 15, i32).at[flat].set(strow)
    c_dst = jnp.full((16 * NCHM * KG,), SAC, i32).at[flat].set(vdst)
    combo = jnp.stack([c_src.reshape(16, NCHM, KG),
                       c_trow.reshape(16, NCHM, KG),
                       c_dst.reshape(16, NCHM, KG)], axis=2)

    kb = jnp.arange(15, dtype=i32)
    bpos = (kb + 1) * _S
    bnode = sdst[jnp.clip(bpos, 0, _M - 1)]
    issplit = (rowptr[bnode] < bpos) & (rowptr[bnode + 1] > bpos)

    z_d = jnp.zeros((RB, _D), jnp.float32)

    i1 = jnp.array([0, 0, 0, 1, 1, 1, 2, 2, 2], i32)
    i2 = jnp.array([0, 1, 2, 0, 1, 2, 0, 1, 2], i32)
    tbl = jnp.concatenate(
        [ee1[:, i1, :] + ee2[:, i2, :],
         (ee1[:, 4, :] + ee2[:, 0, :])[:, None, :],
         jnp.zeros((L, 6, _D), jnp.float32)], axis=1)

    h = (xe1[x[:, 0]] + xe2[x[:, 1]] + xe3[x[:, 2]]
         + xe4[x[:, 3]] + xe5[x[:, 4]] + xe6[x[:, 5]])

    for l in range(L):
        afull = _agg_call(h, tbl[l], combo, z_d)
        aggr = afull[:_N]
        clones = jnp.where(issplit[:, None], afull[_N:_N + 15], 0.0)
        aggr = aggr.at[bnode].add(clones)
        hid = jax.nn.relu(aggr @ W1[l] + b1[l])
        hnew = hid @ W2[l] + b2[l]
        mean = jnp.mean(hnew, axis=0)
        var = jnp.var(hnew, axis=0)
        hnew = (hnew - mean) / jnp.sqrt(var + 1e-5) * gamma[l] + beta[l]
        if l < L - 1:
            hnew = jax.nn.relu(hnew)
        h = hnew
    return h
